# baseline (device time: 18457 ns/iter reference)
import jax
import jax.numpy as jnp
from jax import lax
from jax.experimental import pallas as pl
from jax.experimental.pallas import tpu as pltpu

N_DEV = 4


def kernel(x, w_mat):
    m_per, k = x.shape
    _, n_per = w_mat.shape
    half = m_per // 2

    def body(x_ref, w_ref, out_ref,
             xb, wb, t_m1, b_m1, t_p1, b_p1, t_m2, b_p2,
             send_sems, recv_sems):
        my_pos = lax.axis_index("i")
        left = (my_pos - 1) % N_DEV
        right = (my_pos + 1) % N_DEV

        barrier_sem = pltpu.get_barrier_semaphore()
        for nbr in (left, right):
            pl.semaphore_signal(
                barrier_sem, inc=1,
                device_id=(nbr,), device_id_type=pl.DeviceIdType.MESH,
            )
        xb[:, :] = x_ref[:, :].astype(jnp.bfloat16)
        wb[:, :] = w_ref[:, :].astype(jnp.bfloat16)
        pl.semaphore_wait(barrier_sem, 2)

        def rc(i, src, dst, tgt):
            return pltpu.make_async_remote_copy(
                src_ref=src, dst_ref=dst,
                send_sem=send_sems.at[i], recv_sem=recv_sems.at[i],
                device_id=(tgt,), device_id_type=pl.DeviceIdType.MESH,
            )

        x_top = xb.at[pl.ds(0, half), :]
        x_bot = xb.at[pl.ds(half, half), :]
        d1 = rc(0, x_top, t_m1, right)
        d2 = rc(1, x_bot, b_m1, right)
        d3 = rc(2, x_top, t_p1, left)
        d4 = rc(3, x_bot, b_p1, left)
        d1.start()
        d3.start()
        d2.start()
        d4.start()

        def gemm(buf, origin, off):
            out_ref[pl.ds(origin * m_per + off, half), :] = jnp.dot(
                buf[:, :], wb[:, :], preferred_element_type=jnp.float32,
            )

        out_ref[pl.ds(my_pos * m_per, m_per), :] = jnp.dot(
            xb[:, :], wb[:, :], preferred_element_type=jnp.float32,
        )

        d1.wait_recv()
        d5 = rc(4, t_m1, t_m2, right)
        d5.start()
        d3.wait_recv()
        gemm(t_m1, (my_pos - 1) % N_DEV, 0)
        gemm(t_p1, (my_pos + 1) % N_DEV, 0)

        d2.wait_recv()
        d4.wait_recv()
        d6 = rc(5, b_p1, b_p2, left)
        d6.start()
        gemm(b_m1, (my_pos - 1) % N_DEV, half)
        gemm(b_p1, (my_pos + 1) % N_DEV, half)

        d5.wait_recv()
        gemm(t_m2, (my_pos + 2) % N_DEV, 0)
        d6.wait_recv()
        gemm(b_p2, (my_pos + 2) % N_DEV, half)

        for d in (d1, d2, d3, d4, d5, d6):
            d.wait_send()

    hbuf = lambda: pltpu.VMEM((half, k), jnp.bfloat16)
    return pl.pallas_call(
        body,
        out_shape=jax.ShapeDtypeStruct((N_DEV * m_per, n_per), jnp.float32),
        in_specs=[
            pl.BlockSpec(memory_space=pltpu.VMEM),
            pl.BlockSpec(memory_space=pltpu.VMEM),
        ],
        out_specs=pl.BlockSpec(memory_space=pltpu.VMEM),
        scratch_shapes=[
            pltpu.VMEM((m_per, k), jnp.bfloat16),
            pltpu.VMEM((k, n_per), jnp.bfloat16),
            hbuf(), hbuf(), hbuf(), hbuf(), hbuf(), hbuf(),
            pltpu.SemaphoreType.DMA((6,)),
            pltpu.SemaphoreType.DMA((6,)),
        ],
        compiler_params=pltpu.CompilerParams(collective_id=0),
    )(x, w_mat)


# device time: 17490 ns/iter; 1.0553x vs baseline; 1.0553x over previous
import jax
import jax.numpy as jnp
from jax import lax
from jax.experimental import pallas as pl
from jax.experimental.pallas import tpu as pltpu

N_DEV = 4


def kernel(x, w_mat):
    m_per, k = x.shape
    _, n_per = w_mat.shape
    half = m_per // 2

    def body(x_ref, w_ref, out_ref,
             xb, wb, t_m1, b_m1, t_p1, b_p1, t_m2, b_p2,
             send_sems, recv_sems):
        my_pos = lax.axis_index("i")
        left = (my_pos - 1) % N_DEV
        right = (my_pos + 1) % N_DEV

        barrier_sem = pltpu.get_barrier_semaphore()
        for nbr in (left, right):
            pl.semaphore_signal(
                barrier_sem, inc=1,
                device_id=(nbr,), device_id_type=pl.DeviceIdType.MESH,
            )
        xb[:, :] = x_ref[:, :].astype(jnp.bfloat16)
        wb[:, :] = w_ref[:, :].astype(jnp.bfloat16)
        pl.semaphore_wait(barrier_sem, 2)

        def rc(i, src, dst, tgt):
            return pltpu.make_async_remote_copy(
                src_ref=src, dst_ref=dst,
                send_sem=send_sems.at[i], recv_sem=recv_sems.at[i],
                device_id=(tgt,), device_id_type=pl.DeviceIdType.MESH,
            )

        x_top = xb.at[pl.ds(0, half), :]
        x_bot = xb.at[pl.ds(half, half), :]
        d1 = rc(0, x_top, t_m1, right)
        d2 = rc(1, x_bot, b_m1, right)
        d3 = rc(2, x_top, t_p1, left)
        d4 = rc(3, x_bot, b_p1, left)
        d1.start()
        d4.start()
        d2.start()
        d3.start()

        def gemm(buf, origin, off, rows=half):
            out_ref[pl.ds(origin * m_per + off, rows), :] = jnp.dot(
                buf[...], wb[:, :], preferred_element_type=jnp.float32,
            )

        out_ref[pl.ds(my_pos * m_per, m_per), :] = jnp.dot(
            xb[:, :], wb[:, :], preferred_element_type=jnp.float32,
        )

        sub = half // 2
        d1.wait_recv()
        d5a = rc(4, t_m1.at[pl.ds(0, sub), :], t_m2.at[pl.ds(0, sub), :], right)
        d5b = rc(5, t_m1.at[pl.ds(sub, sub), :], t_m2.at[pl.ds(sub, sub), :], right)
        d5a.start()
        d5b.start()
        d4.wait_recv()
        d6a = rc(6, b_p1.at[pl.ds(0, sub), :], b_p2.at[pl.ds(0, sub), :], left)
        d6b = rc(7, b_p1.at[pl.ds(sub, sub), :], b_p2.at[pl.ds(sub, sub), :], left)
        d6a.start()
        d6b.start()
        gemm(t_m1, (my_pos - 1) % N_DEV, 0)
        gemm(b_p1, (my_pos + 1) % N_DEV, half)

        d2.wait_recv()
        d3.wait_recv()
        gemm(b_m1, (my_pos - 1) % N_DEV, half)
        gemm(t_p1, (my_pos + 1) % N_DEV, 0)

        diag = (my_pos + 2) % N_DEV
        d5a.wait_recv()
        gemm(t_m2.at[pl.ds(0, sub), :], diag, 0, sub)
        d6a.wait_recv()
        gemm(b_p2.at[pl.ds(0, sub), :], diag, half, sub)
        d5b.wait_recv()
        gemm(t_m2.at[pl.ds(sub, sub), :], diag, sub, sub)
        d6b.wait_recv()
        gemm(b_p2.at[pl.ds(sub, sub), :], diag, half + sub, sub)

        for d in (d1, d2, d3, d4, d5a, d5b, d6a, d6b):
            d.wait_send()

    hbuf = lambda: pltpu.VMEM((half, k), jnp.bfloat16)
    return pl.pallas_call(
        body,
        out_shape=jax.ShapeDtypeStruct((N_DEV * m_per, n_per), jnp.float32),
        in_specs=[
            pl.BlockSpec(memory_space=pltpu.VMEM),
            pl.BlockSpec(memory_space=pltpu.VMEM),
        ],
        out_specs=pl.BlockSpec(memory_space=pltpu.VMEM),
        scratch_shapes=[
            pltpu.VMEM((m_per, k), jnp.bfloat16),
            pltpu.VMEM((k, n_per), jnp.bfloat16),
            hbuf(), hbuf(), hbuf(), hbuf(), hbuf(), hbuf(),
            pltpu.SemaphoreType.DMA((8,)),
            pltpu.SemaphoreType.DMA((8,)),
        ],
        compiler_params=pltpu.CompilerParams(collective_id=0),
    )(x, w_mat)
